# Initial kernel scaffold; baseline (speedup 1.0000x reference)
import jax
import jax.numpy as jnp
from jax.experimental import pallas as pl


def kernel(x, gate_w, gate_b, noise_w, noise_b):
    raise NotImplementedError("write your pallas kernel here")


# speedup vs baseline: 2.1591x; 2.1591x over previous
import functools

import jax
import jax.numpy as jnp
from jax import lax
from jax.experimental import pallas as pl
from jax.experimental.pallas import tpu as pltpu
from jax.experimental.pallas import tpu_sc as plsc

_N_TOK = 16384
_D = 2048
_NE = 16
_TOK_BLK = 1024

_NW = 32
_TPW = _N_TOK // _NW
_GRP = _TPW // 16


def _logits_body(x_ref, w_ref, b_ref, o_ref):
    o_ref[...] = lax.dot_general(
        w_ref[...], x_ref[...], (((1,), (1,)), ((), ())),
        preferred_element_type=jnp.float32,
    ) + b_ref[...]


def _compute_logits_t(x, gate_w, gate_b):
    nb = _N_TOK // _TOK_BLK
    return pl.pallas_call(
        _logits_body,
        grid=(nb,),
        in_specs=[
            pl.BlockSpec((_TOK_BLK, _D), lambda i: (i, 0)),
            pl.BlockSpec((_NE, _D), lambda i: (0, 0)),
            pl.BlockSpec((_NE, 1), lambda i: (0, 0)),
        ],
        out_specs=pl.BlockSpec((_NE, _TOK_BLK), lambda i: (0, i)),
        out_shape=jax.ShapeDtypeStruct((_NE, _N_TOK), jnp.float32),
    )(x, gate_w, gate_b.reshape(_NE, 1))


def _routing_body(lt_hbm, probs_hbm, idx_hbm, lt_v, probs_v, idx_v):
    c = lax.axis_index("c")
    s = lax.axis_index("s")
    wid = s * 2 + c
    base = wid * _TPW
    pltpu.sync_copy(lt_hbm.at[:, pl.ds(base, _TPW)], lt_v)

    lanes = lax.iota(jnp.int32, 16)
    neg_inf = jnp.full((16,), -jnp.inf, jnp.float32)
    zeros_f = jnp.zeros((16,), jnp.float32)

    def grp(g, carry):
        off = g * 16
        rows = [lt_v[e, pl.ds(off, 16)] for e in range(_NE)]
        m1 = rows[0]
        a1 = jnp.zeros((16,), jnp.int32)
        for e in range(1, _NE):
            upd = rows[e] > m1
            m1 = jnp.where(upd, rows[e], m1)
            a1 = jnp.where(upd, e, a1)
        m2 = neg_inf
        a2 = jnp.zeros((16,), jnp.int32)
        for e in range(_NE):
            v = jnp.where(a1 == e, neg_inf, rows[e])
            upd = v > m2
            m2 = jnp.where(upd, v, m2)
            a2 = jnp.where(upd, e, a2)
        t = jnp.exp(m2 - m1)
        denom = 1.0 + t
        p1 = 1.0 / denom
        p2 = t / denom
        toks = off + lanes
        pbase = toks * _NE
        for e in range(_NE):
            vals = jnp.where(a1 == e, p1, jnp.where(a2 == e, p2, zeros_f))
            plsc.store_scatter(probs_v, [pbase + e], vals)
        ibase = toks * 2
        plsc.store_scatter(idx_v, [ibase], a1)
        plsc.store_scatter(idx_v, [ibase + 1], a2)
        return carry

    lax.fori_loop(0, _GRP, grp, 0)

    pltpu.sync_copy(probs_v, probs_hbm.at[pl.ds(base * _NE, _TPW * _NE)])
    pltpu.sync_copy(idx_v, idx_hbm.at[pl.ds(base * 2, _TPW * 2)])


@functools.cache
def _make_routing():
    return pl.kernel(
        _routing_body,
        mesh=plsc.VectorSubcoreMesh(core_axis_name="c", subcore_axis_name="s"),
        out_type=[
            jax.ShapeDtypeStruct((_N_TOK * _NE,), jnp.float32),
            jax.ShapeDtypeStruct((_N_TOK * 2,), jnp.int32),
        ],
        scratch_types=[
            pltpu.VMEM((_NE, _TPW), jnp.float32),
            pltpu.VMEM((_TPW * _NE,), jnp.float32),
            pltpu.VMEM((_TPW * 2,), jnp.int32),
        ],
        compiler_params=pltpu.CompilerParams(needs_layout_passes=False),
    )


def kernel(x, gate_w, gate_b, noise_w, noise_b):
    logits_t = _compute_logits_t(x, gate_w, gate_b)
    probs_flat, idx_flat = _make_routing()(logits_t)
    return probs_flat.reshape(_N_TOK, _NE), idx_flat.reshape(_N_TOK, 2)
